# baseline (speedup 1.0000x reference)
import functools

import jax
import jax.numpy as jnp
from jax import lax
from jax.experimental import pallas as pl
from jax.experimental.pallas import tpu as pltpu
from jax.experimental.pallas import tpu_sc as plsc

K = 8192
D = 256
N = 8192
TT = 256
CK = 1024
NTILES = N // TT
NCHUNK = K // CK
COMMITMENT = 0.25

NC = 2
NS = 16
NW = NC * NS
RPW = N // NW
HALF = RPW // 2


STRIPS = (0, 4096, 8192)
NSTRIP = len(STRIPS) - 1


def _argmin_kernel(x_ref, w_ref, idx_ref, enc_ref, loss_ref, perp_ref,
                   acc_ref, hist_ref, bsum_ref):
    t = pl.program_id(0)
    x = x_ref[...]
    s = jnp.sum(x * x, axis=1, keepdims=True)

    @pl.when(t == 0)
    def _():
        for j in range(NCHUNK):
            w = w_ref[j * CK:(j + 1) * CK, :]
            bsum_ref[0, j * CK:(j + 1) * CK] = jnp.sum(w * w, axis=1)

    smin = [jnp.full((TT, 1), jnp.inf, jnp.float32) for _ in range(NSTRIP)]
    sidx = [jnp.zeros((TT, 1), jnp.int32) for _ in range(NSTRIP)]
    for j in range(NCHUNK):
        a_col = j * CK
        w = w_ref[j * CK:(j + 1) * CK, :]
        b = bsum_ref[0, j * CK:(j + 1) * CK]
        c = lax.dot_general(x, w, (((1,), (1,)), ((), ())),
                            preferred_element_type=jnp.float32)
        d = (s + b[None, :]) - 2.0 * c
        k = a_col // STRIPS[1]
        cols = lax.broadcasted_iota(jnp.int32, (TT, CK), 1)
        lmin = jnp.min(d, axis=1, keepdims=True)
        lidx = jnp.min(jnp.where(d == lmin, cols, K),
                       axis=1, keepdims=True) + a_col
        better = lmin < smin[k]
        smin[k] = jnp.where(better, lmin, smin[k])
        sidx[k] = jnp.where(better, lidx, sidx[k])

    accv = smin[0].astype(jnp.bfloat16).astype(jnp.float32)
    acci = sidx[0]
    dsel = smin[0]
    for k in range(1, NSTRIP):
        upd = (smin[k] < accv) | ((smin[k] == accv) & (sidx[k] < acci))
        acci = jnp.where(upd, sidx[k], acci)
        dsel = jnp.where(upd, smin[k], dsel)
        accv = jnp.where(upd, smin[k], accv).astype(jnp.bfloat16).astype(jnp.float32)
    gidx = acci
    gmin = dsel

    idx_ref[0, :, :] = gidx

    cols_full = lax.broadcasted_iota(jnp.int32, (TT, K), 1)
    enc = jnp.where(cols_full == gidx, 1.0, 0.0).astype(jnp.float32)
    enc_ref[...] = enc

    partial_hist = jnp.sum(enc, axis=0, keepdims=True)
    partial_acc = jnp.sum(gmin, axis=0, keepdims=True)

    @pl.when(t == 0)
    def _():
        hist_ref[...] = partial_hist
        acc_ref[...] = partial_acc

    @pl.when(t > 0)
    def _():
        hist_ref[...] = hist_ref[...] + partial_hist
        acc_ref[...] = acc_ref[...] + partial_acc

    @pl.when(t == NTILES - 1)
    def _():
        m = acc_ref[...] * (1.0 / (N * D))
        loss_ref[...] = m + COMMITMENT * m
        avg = hist_ref[...] * (1.0 / N)
        ent = jnp.sum(avg * jnp.log(avg + 1e-10), axis=1, keepdims=True)
        perp_ref[...] = jnp.exp(-ent)


_argmin_call = pl.pallas_call(
    _argmin_kernel,
    grid=(NTILES,),
    in_specs=[
        pl.BlockSpec((TT, D), lambda t: (t, 0)),
        pl.BlockSpec((K, D), lambda t: (0, 0)),
    ],
    out_specs=[
        pl.BlockSpec((1, TT, 1), lambda t: (t, 0, 0)),
        pl.BlockSpec((TT, K), lambda t: (t, 0)),
        pl.BlockSpec((1, 1), lambda t: (0, 0)),
        pl.BlockSpec((1, 1), lambda t: (0, 0)),
    ],
    out_shape=[
        jax.ShapeDtypeStruct((NTILES, TT, 1), jnp.int32),
        jax.ShapeDtypeStruct((N, K), jnp.float32),
        jax.ShapeDtypeStruct((1, 1), jnp.float32),
        jax.ShapeDtypeStruct((1, 1), jnp.float32),
    ],
    scratch_shapes=[
        pltpu.VMEM((1, 1), jnp.float32),
        pltpu.VMEM((1, K), jnp.float32),
        pltpu.VMEM((1, K), jnp.float32),
    ],
    compiler_params=pltpu.CompilerParams(
        dimension_semantics=("arbitrary",),
    ),
)


@functools.cache
def _sc_gather_call():
    mesh = plsc.VectorSubcoreMesh(
        core_axis_name="c", subcore_axis_name="s", num_cores=NC)

    @functools.partial(
        pl.kernel,
        mesh=mesh,
        out_type=jax.ShapeDtypeStruct((N, D), jnp.float32),
        scratch_types=[
            pltpu.VMEM((HALF,), jnp.int32),
            pltpu.VMEM((HALF, D), jnp.float32),
            pltpu.SemaphoreType.DMA,
        ],
    )
    def _sc_gather(idx_hbm, w_hbm, out_hbm, idx_v, rows_v, sem):
        wid = lax.axis_index("s") * NC + lax.axis_index("c")
        base = wid * RPW
        for half in range(RPW // HALF):
            hb = base + half * HALF
            pltpu.sync_copy(idx_hbm.at[pl.ds(hb, HALF)], idx_v)
            pltpu.async_copy(w_hbm.at[idx_v], rows_v, sem).wait()
            pltpu.sync_copy(rows_v, out_hbm.at[pl.ds(hb, HALF), :])

    return _sc_gather


def _st_kernel(x_ref, q_ref, o_ref):
    x = x_ref[...]
    q = q_ref[...]
    o_ref[...] = x + (q - x)


_st_call = pl.pallas_call(
    _st_kernel,
    grid=(NTILES,),
    in_specs=[
        pl.BlockSpec((TT, D), lambda t: (t, 0)),
        pl.BlockSpec((TT, D), lambda t: (t, 0)),
    ],
    out_specs=pl.BlockSpec((TT, D), lambda t: (t, 0)),
    out_shape=jax.ShapeDtypeStruct((N, D), jnp.float32),
)


def kernel(data, weight):
    input_shape = data.shape
    x = data.reshape(N, D)
    idx3, enc, loss, perp = _argmin_call(x, weight)
    idx = idx3.reshape(N)
    q = _sc_gather_call()(idx, weight)
    qst = _st_call(x, q)
    return (qst.reshape(input_shape),
            enc.reshape(tuple(input_shape[:-1]) + (K,)),
            loss[0, 0],
            perp[0, 0])


# speedup vs baseline: 1.0516x; 1.0516x over previous
import functools

import jax
import jax.numpy as jnp
from jax import lax
from jax.experimental import pallas as pl
from jax.experimental.pallas import tpu as pltpu
from jax.experimental.pallas import tpu_sc as plsc

K = 8192
D = 256
N = 8192
TT = 256
CK = 1024
NTILES = N // TT
NCHUNK = K // CK
COMMITMENT = 0.25

NC = 2
NS = 16
NW = NC * NS
RPW = N // NW
HALF = RPW // 2


STRIPS = (0, 4096, 8192)
NSTRIP = len(STRIPS) - 1


def _argmin_kernel(x_ref, w_ref, idx_ref, enc_ref, loss_ref, perp_ref,
                   acc_ref, hist_ref, bsum_ref):
    t = pl.program_id(0)
    x = x_ref[...]
    s = jnp.sum(x * x, axis=1, keepdims=True)
    x2 = 2.0 * x

    @pl.when(t == 0)
    def _():
        for j in range(NCHUNK):
            w = w_ref[j * CK:(j + 1) * CK, :]
            bsum_ref[0, j * CK:(j + 1) * CK] = jnp.sum(w * w, axis=1)

    smin = [jnp.full((TT, 1), jnp.inf, jnp.float32) for _ in range(NSTRIP)]
    sidx = [jnp.zeros((TT, 1), jnp.int32) for _ in range(NSTRIP)]
    for j in range(NCHUNK):
        a_col = j * CK
        w = w_ref[j * CK:(j + 1) * CK, :]
        b = bsum_ref[0, j * CK:(j + 1) * CK]
        c2 = lax.dot_general(x2, w, (((1,), (1,)), ((), ())),
                             preferred_element_type=jnp.float32)
        d = (s + b[None, :]) - c2
        k = a_col // STRIPS[1]
        cols = lax.broadcasted_iota(jnp.int32, (TT, CK), 1)
        lmin = jnp.min(d, axis=1, keepdims=True)
        lidx = jnp.min(jnp.where(d == lmin, cols, K),
                       axis=1, keepdims=True) + a_col
        better = lmin < smin[k]
        smin[k] = jnp.where(better, lmin, smin[k])
        sidx[k] = jnp.where(better, lidx, sidx[k])

    accv = smin[0].astype(jnp.bfloat16).astype(jnp.float32)
    acci = sidx[0]
    dsel = smin[0]
    for k in range(1, NSTRIP):
        upd = (smin[k] < accv) | ((smin[k] == accv) & (sidx[k] < acci))
        acci = jnp.where(upd, sidx[k], acci)
        dsel = jnp.where(upd, smin[k], dsel)
        accv = jnp.where(upd, smin[k], accv).astype(jnp.bfloat16).astype(jnp.float32)
    gidx = acci
    gmin = dsel

    idx_ref[0, :, :] = gidx

    cols_full = lax.broadcasted_iota(jnp.int32, (TT, K), 1)
    enc = jnp.where(cols_full == gidx, 1.0, 0.0).astype(jnp.float32)
    enc_ref[...] = enc

    partial_hist = jnp.sum(enc, axis=0, keepdims=True)
    partial_acc = jnp.sum(gmin, axis=0, keepdims=True)

    @pl.when(t == 0)
    def _():
        hist_ref[...] = partial_hist
        acc_ref[...] = partial_acc

    @pl.when(t > 0)
    def _():
        hist_ref[...] = hist_ref[...] + partial_hist
        acc_ref[...] = acc_ref[...] + partial_acc

    @pl.when(t == NTILES - 1)
    def _():
        m = acc_ref[...] * (1.0 / (N * D))
        loss_ref[...] = m + COMMITMENT * m
        avg = hist_ref[...] * (1.0 / N)
        ent = jnp.sum(avg * jnp.log(avg + 1e-10), axis=1, keepdims=True)
        perp_ref[...] = jnp.exp(-ent)


_argmin_call = pl.pallas_call(
    _argmin_kernel,
    grid=(NTILES,),
    in_specs=[
        pl.BlockSpec((TT, D), lambda t: (t, 0)),
        pl.BlockSpec((K, D), lambda t: (0, 0)),
    ],
    out_specs=[
        pl.BlockSpec((1, TT, 1), lambda t: (t, 0, 0)),
        pl.BlockSpec((TT, K), lambda t: (t, 0)),
        pl.BlockSpec((1, 1), lambda t: (0, 0)),
        pl.BlockSpec((1, 1), lambda t: (0, 0)),
    ],
    out_shape=[
        jax.ShapeDtypeStruct((NTILES, TT, 1), jnp.int32),
        jax.ShapeDtypeStruct((N, K), jnp.float32),
        jax.ShapeDtypeStruct((1, 1), jnp.float32),
        jax.ShapeDtypeStruct((1, 1), jnp.float32),
    ],
    scratch_shapes=[
        pltpu.VMEM((1, 1), jnp.float32),
        pltpu.VMEM((1, K), jnp.float32),
        pltpu.VMEM((1, K), jnp.float32),
    ],
    compiler_params=pltpu.CompilerParams(
        dimension_semantics=("arbitrary",),
    ),
)


@functools.cache
def _sc_gather_call():
    mesh = plsc.VectorSubcoreMesh(
        core_axis_name="c", subcore_axis_name="s", num_cores=NC)

    @functools.partial(
        pl.kernel,
        mesh=mesh,
        out_type=jax.ShapeDtypeStruct((N, D), jnp.float32),
        scratch_types=[
            pltpu.VMEM((HALF,), jnp.int32),
            pltpu.VMEM((HALF, D), jnp.float32),
            pltpu.SemaphoreType.DMA,
        ],
    )
    def _sc_gather(idx_hbm, w_hbm, out_hbm, idx_v, rows_v, sem):
        wid = lax.axis_index("s") * NC + lax.axis_index("c")
        base = wid * RPW
        for half in range(RPW // HALF):
            hb = base + half * HALF
            pltpu.sync_copy(idx_hbm.at[pl.ds(hb, HALF)], idx_v)
            pltpu.async_copy(w_hbm.at[idx_v], rows_v, sem).wait()
            pltpu.sync_copy(rows_v, out_hbm.at[pl.ds(hb, HALF), :])

    return _sc_gather


def _st_kernel(x_ref, q_ref, o_ref):
    x = x_ref[...]
    q = q_ref[...]
    o_ref[...] = x + (q - x)


_st_call = pl.pallas_call(
    _st_kernel,
    grid=(NTILES,),
    in_specs=[
        pl.BlockSpec((TT, D), lambda t: (t, 0)),
        pl.BlockSpec((TT, D), lambda t: (t, 0)),
    ],
    out_specs=pl.BlockSpec((TT, D), lambda t: (t, 0)),
    out_shape=jax.ShapeDtypeStruct((N, D), jnp.float32),
)


def kernel(data, weight):
    input_shape = data.shape
    x = data.reshape(N, D)
    idx3, enc, loss, perp = _argmin_call(x, weight)
    idx = idx3.reshape(N)
    q = _sc_gather_call()(idx, weight)
    qst = _st_call(x, q)
    return (qst.reshape(input_shape),
            enc.reshape(tuple(input_shape[:-1]) + (K,)),
            loss[0, 0],
            perp[0, 0])
